# Initial kernel scaffold; baseline (speedup 1.0000x reference)
import jax
import jax.numpy as jnp
from jax.experimental import pallas as pl


def kernel(x, edge_emb, edge_index):
    raise NotImplementedError("write your pallas kernel here")


# speedup vs baseline: 1.6916x; 1.6916x over previous
import functools

import jax
import jax.numpy as jnp
from jax import lax
from jax.experimental import pallas as pl
from jax.experimental.pallas import tpu as pltpu
from jax.experimental.pallas import tpu_sc as plsc

NC, NS = 2, 16
NW = NC * NS
E = 320000
CH = 128
N_CHUNKS = E // CH
FULL = N_CHUNKS // NW
REM = N_CHUNKS - FULL * NW
D = 128
DE = 16
DO = D + DE + D

_mesh = plsc.VectorSubcoreMesh(
    core_axis_name="c", subcore_axis_name="s", num_cores=NC, num_subcores=NS
)


@functools.partial(
    pl.kernel,
    out_type=jax.ShapeDtypeStruct((E, DO), jnp.float32),
    mesh=_mesh,
    scratch_types=[
        pltpu.VMEM((CH,), jnp.int32),
        pltpu.VMEM((CH,), jnp.int32),
        pltpu.VMEM((CH, D), jnp.float32),
        pltpu.VMEM((CH, D), jnp.float32),
        pltpu.VMEM((CH, DE), jnp.float32),
        pltpu.SemaphoreType.DMA,
        pltpu.SemaphoreType.DMA,
    ],
    compiler_params=pltpu.CompilerParams(use_tc_tiling_on_sc=False),
)
def _triplet_cat_sc(x_hbm, edge_hbm, src_hbm, dst_hbm, out_hbm,
                    sidx, didx, srows, drows, erows, sem_s, sem_d):
    wid = lax.axis_index("s") * NC + lax.axis_index("c")

    def do_chunk(chunk_id):
        base = chunk_id * CH
        pltpu.sync_copy(src_hbm.at[pl.ds(base, CH)], sidx)
        pltpu.sync_copy(dst_hbm.at[pl.ds(base, CH)], didx)
        cs = pltpu.async_copy(x_hbm.at[sidx], srows, sem_s)
        cd = pltpu.async_copy(x_hbm.at[didx], drows, sem_d)
        pltpu.sync_copy(edge_hbm.at[pl.ds(base, CH)], erows)
        cs.wait()
        cd.wait()
        pltpu.sync_copy(srows, out_hbm.at[pl.ds(base, CH), pl.ds(0, D)])
        pltpu.sync_copy(erows, out_hbm.at[pl.ds(base, CH), pl.ds(D, DE)])
        pltpu.sync_copy(drows, out_hbm.at[pl.ds(base, CH), pl.ds(D + DE, D)])

    def body(j, carry):
        do_chunk(wid * FULL + j)
        return carry

    lax.fori_loop(0, FULL, body, 0)

    @pl.when(wid < REM)
    def _():
        do_chunk(NW * FULL + wid)


def kernel(x, edge_emb, edge_index):
    src = edge_index[0].astype(jnp.int32)
    dst = edge_index[1].astype(jnp.int32)
    return _triplet_cat_sc(x, edge_emb, src, dst)


# speedup vs baseline: 1.8822x; 1.1126x over previous
import functools

import jax
import jax.numpy as jnp
from jax import lax
from jax.experimental import pallas as pl
from jax.experimental.pallas import tpu as pltpu
from jax.experimental.pallas import tpu_sc as plsc

NC, NS = 2, 16
NW = NC * NS
E = 320000
EPW = E // NW
CH = 128
NFULL = EPW // CH
CT = EPW - NFULL * CH
D = 128
DE = 16
DO = D + DE + D
NBUF = 3

_mesh = plsc.VectorSubcoreMesh(
    core_axis_name="c", subcore_axis_name="s", num_cores=NC, num_subcores=NS
)


@functools.partial(
    pl.kernel,
    out_type=jax.ShapeDtypeStruct((E, DO), jnp.float32),
    mesh=_mesh,
    scratch_types=[
        pltpu.VMEM((EPW,), jnp.int32),
        pltpu.VMEM((EPW,), jnp.int32),
        pltpu.VMEM((NBUF, CH, D), jnp.float32),
        pltpu.VMEM((NBUF, CH, D), jnp.float32),
        pltpu.VMEM((NBUF, CH, DE), jnp.float32),
        pltpu.SemaphoreType.DMA,
        pltpu.SemaphoreType.DMA,
        pltpu.SemaphoreType.DMA,
        pltpu.SemaphoreType.DMA,
        pltpu.SemaphoreType.DMA,
        pltpu.SemaphoreType.DMA,
    ],
    compiler_params=pltpu.CompilerParams(use_tc_tiling_on_sc=False),
)
def _triplet_cat_sc(x_hbm, edge_hbm, src_hbm, dst_hbm, out_hbm,
                    sidx, didx, srows, drows, erows, si0, si1, si2, so0, so1, so2):
    wid = lax.axis_index("s") * NC + lax.axis_index("c")
    wbase = wid * EPW
    sem_in = [si0, si1, si2]
    sem_out = [so0, so1, so2]

    pltpu.sync_copy(src_hbm.at[pl.ds(wbase, EPW)], sidx)
    pltpu.sync_copy(dst_hbm.at[pl.ds(wbase, EPW)], didx)

    def in_copies(lbase, b, n):
        return (
            pltpu.make_async_copy(
                x_hbm.at[sidx.at[pl.ds(lbase, n)]],
                srows.at[b].at[pl.ds(0, n)], sem_in[b]),
            pltpu.make_async_copy(
                edge_hbm.at[pl.ds(wbase + lbase, n)],
                erows.at[b].at[pl.ds(0, n)], sem_in[b]),
            pltpu.make_async_copy(
                x_hbm.at[didx.at[pl.ds(lbase, n)]],
                drows.at[b].at[pl.ds(0, n)], sem_in[b]),
        )

    def out_copies(lbase, b, n):
        rows = out_hbm.at[pl.ds(wbase + lbase, n)]
        return (
            pltpu.make_async_copy(
                srows.at[b].at[pl.ds(0, n)], rows.at[:, pl.ds(0, D)], sem_out[b]),
            pltpu.make_async_copy(
                erows.at[b].at[pl.ds(0, n)], rows.at[:, pl.ds(D, DE)], sem_out[b]),
            pltpu.make_async_copy(
                drows.at[b].at[pl.ds(0, n)], rows.at[:, pl.ds(D + DE, D)], sem_out[b]),
        )

    for b in range(NBUF):
        for c in in_copies(b * CH, b, CH):
            c.start()

    def body(g, carry):
        for b in range(NBUF):
            j = g + b
            lbase = j * CH
            for c in in_copies(lbase, b, CH):
                c.wait()
            for c in out_copies(lbase, b, CH):
                c.start()
            for c in out_copies(lbase, b, CH):
                c.wait()

            @pl.when(j + NBUF < NFULL)
            def _():
                for c in in_copies((j + NBUF) * CH, b, CH):
                    c.start()
        return carry

    lax.fori_loop(0, NFULL // NBUF, lambda g, c: body(g * NBUF, c), 0,
                  unroll=False)

    tbase = NFULL * CH
    for c in in_copies(tbase, 0, CT):
        c.start()
    for c in in_copies(tbase, 0, CT):
        c.wait()
    for c in out_copies(tbase, 0, CT):
        c.start()
    for c in out_copies(tbase, 0, CT):
        c.wait()


def kernel(x, edge_emb, edge_index):
    src = edge_index[0].astype(jnp.int32)
    dst = edge_index[1].astype(jnp.int32)
    return _triplet_cat_sc(x, edge_emb, src, dst)


# speedup vs baseline: 2.7902x; 1.4824x over previous
import functools

import jax
import jax.numpy as jnp
from jax import lax
from jax.experimental import pallas as pl
from jax.experimental.pallas import tpu as pltpu
from jax.experimental.pallas import tpu_sc as plsc

NC, NS = 2, 16
NW = NC * NS
E = 320000
EPW = E // NW
CH = 64
NFULL = EPW // CH
CT = EPW - NFULL * CH
D = 128
DE = 16
DO = D + DE + D
NBUF = 3
L = 16

_mesh = plsc.VectorSubcoreMesh(
    core_axis_name="c", subcore_axis_name="s", num_cores=NC, num_subcores=NS
)


@functools.partial(
    pl.kernel,
    out_type=jax.ShapeDtypeStruct((E, DO), jnp.float32),
    mesh=_mesh,
    scratch_types=[
        pltpu.VMEM((EPW,), jnp.int32),
        pltpu.VMEM((EPW,), jnp.int32),
    ] + [pltpu.VMEM((CH, D), jnp.float32)] * NBUF
      + [pltpu.VMEM((CH * DE,), jnp.float32)] * NBUF
      + [pltpu.VMEM((CH, DO), jnp.float32)] * NBUF
      + [pltpu.SemaphoreType.DMA] * (2 * NBUF),
    compiler_params=pltpu.CompilerParams(use_tc_tiling_on_sc=True),
)
def _triplet_cat_sc(x_hbm, eflat_hbm, src_hbm, dst_hbm, out_hbm,
                    sidx, didx, dr0, dr1, dr2, ef0, ef1, ef2,
                    rw0, rw1, rw2, si0, si1, si2, so0, so1, so2):
    wid = lax.axis_index("s") * NC + lax.axis_index("c")
    wbase = wid * EPW
    drows = [dr0, dr1, dr2]
    eflat = [ef0, ef1, ef2]
    rows = [rw0, rw1, rw2]
    sem_in = [si0, si1, si2]
    sem_out = [so0, so1, so2]

    pltpu.sync_copy(src_hbm.at[pl.ds(wbase, EPW)], sidx)
    pltpu.sync_copy(dst_hbm.at[pl.ds(wbase, EPW)], didx)

    def in_copies(lbase, b, n):
        return (
            pltpu.make_async_copy(
                x_hbm.at[sidx.at[pl.ds(lbase, n)]],
                rows[b].at[pl.ds(0, n), pl.ds(0, D)], sem_in[b]),
            pltpu.make_async_copy(
                eflat_hbm.at[pl.ds((wbase + lbase) * DE, n * DE)],
                eflat[b].at[pl.ds(0, n * DE)], sem_in[b]),
            pltpu.make_async_copy(
                x_hbm.at[didx.at[pl.ds(lbase, n)]],
                drows[b].at[pl.ds(0, n)], sem_in[b]),
        )

    def out_copy(lbase, b, n):
        return pltpu.make_async_copy(
            rows[b].at[pl.ds(0, n)],
            out_hbm.at[pl.ds(wbase + lbase, n)], sem_out[b])

    def assemble(b, n):
        dr, ef, rw = drows[b], eflat[b], rows[b]
        for r in range(n):
            rw[r, pl.ds(D, L)] = ef[pl.ds(r * DE, L)]
            for k in range(D // L):
                rw[r, pl.ds(D + DE + k * L, L)] = dr[r, pl.ds(k * L, L)]

    for b in range(NBUF):
        for c in in_copies(b * CH, b, CH):
            c.start()

    def body(g, carry):
        for b in range(NBUF):
            j = g + b
            lbase = j * CH
            for c in in_copies(lbase, b, CH):
                c.wait()
            assemble(b, CH)
            out_copy(lbase, b, CH).start()
            out_copy(lbase, b, CH).wait()

            @pl.when(j + NBUF < NFULL)
            def _():
                for c in in_copies((j + NBUF) * CH, b, CH):
                    c.start()
        return carry

    lax.fori_loop(0, NFULL // NBUF, lambda g, c: body(g * NBUF, c), 0,
                  unroll=False)

    tbase = NFULL * CH
    for c in in_copies(tbase, 0, CT):
        c.start()
    for c in in_copies(tbase, 0, CT):
        c.wait()
    assemble(0, CT)
    out_copy(tbase, 0, CT).start()
    out_copy(tbase, 0, CT).wait()


def kernel(x, edge_emb, edge_index):
    src = edge_index[0].astype(jnp.int32)
    dst = edge_index[1].astype(jnp.int32)
    eflat = edge_emb.reshape(-1)
    return _triplet_cat_sc(x, eflat, src, dst)
